# baseline (speedup 1.0000x reference)
import functools

import jax
import jax.numpy as jnp
from jax import lax
from jax.experimental import pallas as pl
from jax.experimental.pallas import tpu as pltpu
from jax.experimental.pallas import tpu_sc as plsc

_MOVIES_NUM = 1000000
_K = 64
_BATCH = 16384
_CHUNK = 128


def _make_kernel():
    info = plsc.get_sparse_core_info()
    nw = info.num_cores * info.num_subcores
    b_per_w = _BATCH // nw
    n_chunks = b_per_w // _CHUNK
    mesh = plsc.VectorSubcoreMesh(core_axis_name="c", subcore_axis_name="s")

    @functools.partial(
        pl.kernel,
        mesh=mesh,
        out_type=(
            jax.ShapeDtypeStruct((_BATCH, _K), jnp.float32),
            jax.ShapeDtypeStruct((_BATCH,), jnp.float32),
        ),
        scratch_types=[
            pltpu.VMEM((n_chunks, _CHUNK), jnp.int32),
            pltpu.VMEM((b_per_w, _K), jnp.float32),
            pltpu.VMEM((b_per_w,), jnp.float32),
            pltpu.SemaphoreType.DMA,
        ],
        compiler_params=pltpu.CompilerParams(use_tc_tiling_on_sc=False),
    )
    def sc_gather(idx_hbm, table_hbm, bias_hbm, emb_out, bias_out,
                  idx_v, rows_v, brows_v, sem):
        wid = lax.axis_index("s") * info.num_cores + lax.axis_index("c")
        base = wid * b_per_w
        pltpu.sync_copy(idx_hbm.at[pl.ds(wid * n_chunks, n_chunks)], idx_v)
        copies = []
        for j in range(n_chunks):
            copies.append(pltpu.async_copy(
                table_hbm.at[idx_v.at[j]],
                rows_v.at[pl.ds(j * _CHUNK, _CHUNK)],
                sem,
            ))
            copies.append(pltpu.async_copy(
                bias_hbm.at[idx_v.at[j]],
                brows_v.at[pl.ds(j * _CHUNK, _CHUNK)],
                sem,
            ))
        for c in copies:
            c.wait()
        pltpu.sync_copy(rows_v, emb_out.at[pl.ds(base, b_per_w)])
        pltpu.sync_copy(brows_v, bias_out.at[pl.ds(base, b_per_w)])

    return sc_gather


_SC_GATHER = _make_kernel()


def kernel(movie_id, movie, bias_movie):
    idx = movie_id.astype(jnp.int32).reshape(_BATCH // _CHUNK, _CHUNK)
    emb, bias = _SC_GATHER(idx, movie, jnp.zeros((_MOVIES_NUM,), jnp.float32))
    return emb, bias.reshape(_BATCH, 1)


# speedup vs baseline: 1.0535x; 1.0535x over previous
import functools

import jax
import jax.numpy as jnp
from jax import lax
from jax.experimental import pallas as pl
from jax.experimental.pallas import tpu as pltpu
from jax.experimental.pallas import tpu_sc as plsc

_MOVIES_NUM = 1000000
_K = 64
_BATCH = 16384


def _make_kernel():
    info = plsc.get_sparse_core_info()
    nc = info.num_cores
    nw = nc * info.num_subcores
    b_per_w = _BATCH // nw
    mesh = plsc.VectorSubcoreMesh(core_axis_name="c", subcore_axis_name="s")

    @functools.partial(
        pl.kernel,
        mesh=mesh,
        out_type=(
            jax.ShapeDtypeStruct((_K, _BATCH), jnp.float32),
            jax.ShapeDtypeStruct((_BATCH,), jnp.float32),
        ),
        scratch_types=[
            pltpu.VMEM((b_per_w,), jnp.int32),
            pltpu.VMEM((_K, 128), jnp.float32),
            pltpu.VMEM((_K, b_per_w), jnp.float32),
            pltpu.VMEM((b_per_w,), jnp.float32),
            pltpu.SemaphoreType.DMA,
        ],
        compiler_params=pltpu.CompilerParams(use_tc_tiling_on_sc=True,
                                             needs_layout_passes=False),
    )
    def sc_gather(idx_hbm, table_t_hbm, bias_hbm, emb_out, bias_out,
                  idx_v, slab_v, outt_v, bias_v, sem):
        wid = lax.axis_index("s") * nc + lax.axis_index("c")
        base = wid * b_per_w
        pltpu.sync_copy(idx_hbm.at[pl.ds(base, b_per_w)], idx_v)
        pltpu.async_copy(bias_hbm.at[idx_v], bias_v, sem).wait()
        pltpu.sync_copy(bias_v, bias_out.at[pl.ds(base, b_per_w)])

        jrow = lax.iota(jnp.int32, 16)

        def body(g, carry):
            iv = idx_v[pl.ds(g * 16, 16)]
            lanes = jnp.bitwise_and(iv, 127)
            cols = iv - lanes
            for e in range(16):
                col = pl.multiple_of(cols[e], 128)
                pltpu.sync_copy(table_t_hbm.at[:, pl.ds(col, 128)], slab_v)
                lv = jnp.full((16,), lanes[e], jnp.int32)
                kv = jnp.full((16,), g * 16 + e, jnp.int32)
                for q in range(4):
                    vals = plsc.load_gather(slab_v, [jrow + q * 16, lv])
                    plsc.store_scatter(outt_v, [jrow + q * 16, kv], vals)
            return carry

        lax.fori_loop(0, b_per_w // 16, body, 0)
        pltpu.sync_copy(outt_v, emb_out.at[:, pl.ds(base, b_per_w)])

    return sc_gather


_SC_GATHER = _make_kernel()


def kernel(movie_id, movie, bias_movie):
    idx = movie_id.astype(jnp.int32)
    emb_t, bias = _SC_GATHER(idx, movie.T, bias_movie.reshape(_MOVIES_NUM))
    return emb_t.T, bias.reshape(_BATCH, 1)


# speedup vs baseline: 2.5190x; 2.3910x over previous
import functools

import jax
import jax.numpy as jnp
from jax import lax
from jax.experimental import pallas as pl
from jax.experimental.pallas import tpu as pltpu
from jax.experimental.pallas import tpu_sc as plsc

_MOVIES_NUM = 1000000
_K = 64
_BATCH = 16384
_NBUF = 8


def _make_kernel():
    info = plsc.get_sparse_core_info()
    nc = info.num_cores
    nw = nc * info.num_subcores
    b_per_w = _BATCH // nw
    n_groups = b_per_w // 16
    mesh = plsc.VectorSubcoreMesh(core_axis_name="c", subcore_axis_name="s")

    @functools.partial(
        pl.kernel,
        mesh=mesh,
        out_type=(
            jax.ShapeDtypeStruct((_K, _BATCH), jnp.float32),
            jax.ShapeDtypeStruct((_BATCH,), jnp.float32),
        ),
        scratch_types=[
            pltpu.VMEM((b_per_w,), jnp.int32),
            pltpu.VMEM((_NBUF, _K, 128), jnp.float32),
            pltpu.VMEM((_K, b_per_w), jnp.float32),
            pltpu.VMEM((b_per_w,), jnp.float32),
            pltpu.SemaphoreType.DMA,
            pltpu.SemaphoreType.DMA((_NBUF,)),
        ],
        compiler_params=pltpu.CompilerParams(use_tc_tiling_on_sc=True,
                                             needs_layout_passes=False),
    )
    def sc_gather(idx_hbm, table_t_hbm, bias_hbm, emb_out, bias_out,
                  idx_v, slab_v, outt_v, bias_v, bsem, sems):
        wid = lax.axis_index("s") * nc + lax.axis_index("c")
        base = wid * b_per_w
        pltpu.sync_copy(idx_hbm.at[pl.ds(base, b_per_w)], idx_v)
        bias_cp = pltpu.async_copy(bias_hbm.at[idx_v], bias_v, bsem)

        jrow = lax.iota(jnp.int32, 16)

        def issue(buf, col_scalar):
            col = pl.multiple_of(col_scalar, 128)
            pltpu.async_copy(table_t_hbm.at[:, pl.ds(col, 128)],
                             slab_v.at[buf], sems.at[buf])

        def drain(buf):
            pltpu.make_async_copy(table_t_hbm.at[:, pl.ds(0, 128)],
                                  slab_v.at[buf], sems.at[buf]).wait()

        iv0 = idx_v[pl.ds(0, 16)]
        lanes0 = jnp.bitwise_and(iv0, 127)
        cols0 = iv0 - lanes0
        for e in range(_NBUF):
            issue(e, cols0[e])

        def body(g, carry):
            lanes_g, cols_g = carry
            off1 = jnp.minimum((g + 1) * 16, (n_groups - 1) * 16)
            iv1 = idx_v[pl.ds(off1, 16)]
            lanes1 = jnp.bitwise_and(iv1, 127)
            cols1 = iv1 - lanes1
            for e in range(16):
                buf = e % _NBUF
                drain(buf)
                lv = jnp.full((16,), lanes_g[e], jnp.int32)
                kv = jnp.full((16,), g * 16 + e, jnp.int32)
                for q in range(4):
                    vals = plsc.load_gather(slab_v.at[buf],
                                            [jrow + q * 16, lv])
                    plsc.store_scatter(outt_v, [jrow + q * 16, kv], vals)
                if e < 16 - _NBUF:
                    issue(buf, cols_g[e + _NBUF])
                else:
                    issue(buf, cols1[e + _NBUF - 16])
            return (lanes1, cols1)

        lax.fori_loop(0, n_groups, body, (lanes0, cols0))
        for e in range(_NBUF):
            drain(e)

        pltpu.sync_copy(outt_v, emb_out.at[:, pl.ds(base, b_per_w)])
        bias_cp.wait()
        pltpu.sync_copy(bias_v, bias_out.at[pl.ds(base, b_per_w)])

    return sc_gather


_SC_GATHER = _make_kernel()


def kernel(movie_id, movie, bias_movie):
    idx = movie_id.astype(jnp.int32)
    emb_t, bias = _SC_GATHER(idx, movie.T, bias_movie.reshape(_MOVIES_NUM))
    return emb_t.T, bias.reshape(_BATCH, 1)
